# baseline (speedup 1.0000x reference)
import functools

import jax
import jax.numpy as jnp
from jax import lax
from jax.experimental import pallas as pl
from jax.experimental.pallas import tpu as pltpu
from jax.experimental.pallas import tpu_sc as plsc

D = 32
G = 4
S = 8
L = 128


@functools.lru_cache(maxsize=None)
def _build(B: int, T: int, V: int):
    info = plsc.get_sparse_core_info()
    NC, NS = info.num_cores, info.num_subcores
    NW = NC * NS
    b_per_w = B // NW
    tc_per_w = b_per_w // L
    n_steps = T * tc_per_w
    assert n_steps % 2 == 0

    mesh = plsc.VectorSubcoreMesh(core_axis_name="c", subcore_axis_name="s")

    @functools.partial(
        pl.kernel,
        mesh=mesh,
        out_type=jax.ShapeDtypeStruct((T, G, B // L, S, L), jnp.float32),
        compiler_params=pltpu.CompilerParams(
            use_tc_tiling_on_sc=False, needs_layout_passes=False),
        scratch_types=[
            pltpu.VMEM((T, b_per_w), jnp.int32),
            pltpu.VMEM((2, L, D), jnp.float32),
            pltpu.VMEM((2, G, S, L), jnp.float32),
            pltpu.SemaphoreType.DMA,
            pltpu.SemaphoreType.DMA,
        ],
    )
    def gather_kernel(idt_hbm, table_hbm, out_hbm, idx_v, rows_v, nat_v,
                      gsem, osem):
        wid = lax.axis_index("s") * NC + lax.axis_index("c")
        tc0 = wid * tc_per_w
        pltpu.sync_copy(idt_hbm.at[:, pl.ds(wid * b_per_w, b_per_w)], idx_v)

        lvecs = [
            jnp.arange(l0, l0 + 16, dtype=jnp.int32) for l0 in range(0, L, 16)
        ]

        def idx_slice(i):
            t, tcl = i // tc_per_w, i % tc_per_w
            return idx_v.at[t, pl.ds(tcl * L, L)]

        def gather(i, b):
            pltpu.async_copy(table_hbm.at[idx_slice(i)], rows_v.at[b], gsem)

        def wait_gather(i, b):
            pltpu.make_async_copy(
                table_hbm.at[idx_slice(i)], rows_v.at[b], gsem).wait()

        def transpose(b):
            src = rows_v.at[b]
            for g in range(G):
                for s in range(S):
                    col = jnp.full((16,), g * S + s, jnp.int32)
                    for k in range(L // 16):
                        v = plsc.load_gather(src, [lvecs[k], col])
                        nat_v[b, g, s, pl.ds(k * 16, 16)] = v

        def write(i, b):
            t, tcl = i // tc_per_w, i % tc_per_w
            for g in range(G):
                pltpu.async_copy(
                    nat_v.at[b, g], out_hbm.at[t, g, tc0 + tcl], osem)

        def drain_writes():
            for _ in range(G):
                pltpu.make_async_copy(
                    nat_v.at[0, 0], out_hbm.at[0, 0, 0], osem).wait()

        gather(0, 0)

        def super_step(p, _):
            for b in range(2):
                i = p * 2 + b

                @pl.when(i >= 2)
                def _():
                    drain_writes()

                @pl.when(i + 1 < n_steps)
                def _():
                    gather(i + 1, 1 - b)

                wait_gather(i, b)
                transpose(b)
                write(i, b)
            return 0

        lax.fori_loop(0, n_steps // 2, super_step, 0)
        drain_writes()
        drain_writes()

    return gather_kernel


def kernel(input_ids, W):
    Bt, T = input_ids.shape
    ids_t = input_ids.T.astype(jnp.int32)
    fn = _build(Bt, T, W.shape[0])
    out5 = fn(ids_t, W)
    return out5.transpose(2, 4, 0, 1, 3).reshape(Bt, T, D)


# speedup vs baseline: 1.2350x; 1.2350x over previous
import functools

import jax
import jax.numpy as jnp
from jax import lax
from jax.experimental import pallas as pl
from jax.experimental.pallas import tpu as pltpu
from jax.experimental.pallas import tpu_sc as plsc

D = 32
G = 4
S = 8
L = 128


@functools.lru_cache(maxsize=None)
def _build(B: int, T: int, V: int):
    info = plsc.get_sparse_core_info()
    NC, NS = info.num_cores, info.num_subcores
    NW = NC * NS
    b_per_w = B // NW
    tc_per_w = b_per_w // L
    n_steps = T * tc_per_w
    assert n_steps % 2 == 0

    mesh = plsc.VectorSubcoreMesh(core_axis_name="c", subcore_axis_name="s")

    @functools.partial(
        pl.kernel,
        mesh=mesh,
        out_type=jax.ShapeDtypeStruct((T, G, B // L, S, L), jnp.float32),
        compiler_params=pltpu.CompilerParams(
            use_tc_tiling_on_sc=False, needs_layout_passes=False),
        scratch_types=[
            pltpu.VMEM((T, b_per_w), jnp.int32),
            pltpu.VMEM((2, L, D), jnp.float32),
            pltpu.VMEM((2, L, D + 1), jnp.float32),
            pltpu.VMEM((2, G, S, L), jnp.float32),
            pltpu.SemaphoreType.DMA,
            pltpu.SemaphoreType.DMA,
        ],
    )
    def gather_kernel(idt_hbm, table_hbm, out_hbm, idx_v, rows_v, pad_v,
                      nat_v, gsem, osem):
        wid = lax.axis_index("s") * NC + lax.axis_index("c")
        tc0 = wid * tc_per_w
        pltpu.sync_copy(idt_hbm.at[:, pl.ds(wid * b_per_w, b_per_w)], idx_v)

        lvecs = [
            jnp.arange(l0, l0 + 16, dtype=jnp.int32) for l0 in range(0, L, 16)
        ]

        def idx_slice(i):
            t, tcl = i // tc_per_w, i % tc_per_w
            return idx_v.at[t, pl.ds(tcl * L, L)]

        def gather(i, b):
            pltpu.async_copy(table_hbm.at[idx_slice(i)], rows_v.at[b], gsem)

        def wait_gather(i, b):
            pltpu.make_async_copy(
                table_hbm.at[idx_slice(i)], rows_v.at[b], gsem).wait()

        def transpose(b):
            for l in range(L):
                for c in range(0, D, 16):
                    pad_v[b, l, pl.ds(c, 16)] = rows_v[b, l, pl.ds(c, 16)]
            src = pad_v.at[b]
            for g in range(G):
                for s in range(S):
                    col = jnp.full((16,), g * S + s, jnp.int32)
                    for k in range(L // 16):
                        v = plsc.load_gather(src, [lvecs[k], col])
                        nat_v[b, g, s, pl.ds(k * 16, 16)] = v

        def write(i, b):
            t, tcl = i // tc_per_w, i % tc_per_w
            for g in range(G):
                pltpu.async_copy(
                    nat_v.at[b, g], out_hbm.at[t, g, tc0 + tcl], osem)

        def drain_writes():
            for _ in range(G):
                pltpu.make_async_copy(
                    nat_v.at[0, 0], out_hbm.at[0, 0, 0], osem).wait()

        gather(0, 0)

        def super_step(p, _):
            for b in range(2):
                i = p * 2 + b

                @pl.when(i >= 2)
                def _():
                    drain_writes()

                @pl.when(i + 1 < n_steps)
                def _():
                    gather(i + 1, 1 - b)

                wait_gather(i, b)
                transpose(b)
                write(i, b)
            return 0

        lax.fori_loop(0, n_steps // 2, super_step, 0)
        drain_writes()
        drain_writes()

    return gather_kernel


def kernel(input_ids, W):
    Bt, T = input_ids.shape
    ids_t = input_ids.T.astype(jnp.int32)
    fn = _build(Bt, T, W.shape[0])
    out5 = fn(ids_t, W)
    return out5.transpose(2, 4, 0, 1, 3).reshape(Bt, T, D)
